# baseline (speedup 1.0000x reference)
import functools
import math

import jax
import jax.numpy as jnp
from jax import lax
from jax.experimental import pallas as pl
from jax.experimental.pallas import tpu as pltpu, tpu_sc as plsc

V = 100000
SMOOTH = 0.1
CONF = 1.0 - SMOOTH
EPS = SMOOTH / (V - 2)
K_ROW = CONF * math.log(CONF) + (V - 2) * EPS * math.log(EPS)

RB = 1024
CB = 2560
N_ROWS = 1024


def _rowsum(x, j_block):
    col = jax.lax.broadcasted_iota(jnp.int32, x.shape, 1) + j_block * CB
    xz = jnp.where(col < V, x, 0.0)
    return jnp.sum(xz, axis=1, keepdims=True)


def _loss_body(tok_ref, xa_ref, xb_ref, out_ref):
    j = pl.program_id(0)
    c = tok_ref[...]
    notpad = (c != 0.0).astype(jnp.float32)
    term = _rowsum(xa_ref[...], 2 * j) + _rowsum(xb_ref[...], 2 * j + 1)
    contrib = jnp.sum(notpad * term) * (-EPS)
    extra = jnp.sum(notpad * (K_ROW + EPS * xa_ref[:, 0:1]))
    contrib = contrib + jnp.where(j == 0, extra, 0.0)

    @pl.when(j == 0)
    def _init():
        out_ref[...] = jnp.zeros((1, 1), jnp.float32)

    out_ref[...] += jnp.full((1, 1), contrib, jnp.float32)


def _tc_pass(tok_col, x):
    grid = (pl.cdiv(V, CB) // 2,)
    out = pl.pallas_call(
        _loss_body,
        grid=grid,
        in_specs=[
            pl.BlockSpec((RB, 1), lambda j: (0, 0)),
            pl.BlockSpec((RB, CB), lambda j: (0, 2 * j)),
            pl.BlockSpec((RB, CB), lambda j: (0, 2 * j + 1)),
        ],
        out_specs=pl.BlockSpec((1, 1), lambda j: (0, 0)),
        out_shape=jax.ShapeDtypeStruct((1, 1), jnp.float32),
    )(tok_col, x, x)
    return out[0, 0]


_SC_INFO = plsc.get_sparse_core_info()
_NC, _NS = _SC_INFO.num_cores, _SC_INFO.num_subcores
_NW = _NC * _NS
_BPW = N_ROWS // _NW

_sc_mesh = plsc.VectorSubcoreMesh(core_axis_name="c", subcore_axis_name="s")


@functools.partial(
    pl.kernel,
    mesh=_sc_mesh,
    out_type=jax.ShapeDtypeStruct((N_ROWS,), jnp.float32),
    scratch_types=[
        pltpu.VMEM((_BPW,), jnp.int32),
        pltpu.VMEM((_BPW,), jnp.float32),
        pltpu.SemaphoreType.DMA,
    ],
)
def _sc_gather(xflat_hbm, flatidx_hbm, out_hbm, idx_v, val_v, sem):
    wid = lax.axis_index("s") * _NC + lax.axis_index("c")
    base = wid * _BPW
    pltpu.sync_copy(flatidx_hbm.at[pl.ds(base, _BPW)], idx_v)
    pltpu.async_copy(xflat_hbm.at[idx_v], val_v, sem).wait()
    pltpu.sync_copy(val_v, out_hbm.at[pl.ds(base, _BPW)])


def kernel(predicted_log_probabilities, tgt_tokens):
    n, v = predicted_log_probabilities.shape
    x = predicted_log_probabilities
    tok_col = tgt_tokens.reshape(n, 1).astype(jnp.float32)
    flat_idx = (jnp.arange(n, dtype=jnp.int32) * v + tgt_tokens).astype(jnp.int32)
    g = _sc_gather(x.reshape(n * v), flat_idx)
    a = _tc_pass(tok_col, x)
    notpad = (tgt_tokens != 0).astype(jnp.float32)
    return a - (CONF - EPS) * jnp.sum(notpad * g)


# speedup vs baseline: 2.1296x; 2.1296x over previous
import math

import jax
import jax.numpy as jnp
from jax.experimental import pallas as pl

V = 100000
SMOOTH = 0.1
CONF = 1.0 - SMOOTH
EPS = SMOOTH / (V - 2)
K_ROW = CONF * math.log(CONF) + (V - 2) * EPS * math.log(EPS)

RB = 128
CB = 12544


def _loss_body(tok_ref, x_ref, out_ref):
    i = pl.program_id(0)
    j = pl.program_id(1)
    x = x_ref[...]
    c = tok_ref[...]
    notpad = (c != 0.0).astype(jnp.float32)
    col = jax.lax.broadcasted_iota(jnp.int32, (RB, CB), 1) + j * CB
    coeff = jnp.where(col.astype(jnp.float32) == c, -CONF, -EPS)
    xz = jnp.where(col < V, x, 0.0)
    term = jnp.sum(coeff * xz, axis=1, keepdims=True)
    contrib = jnp.sum(notpad * term)
    extra = jnp.sum(notpad * (K_ROW + EPS * x[:, 0:1]))
    contrib = contrib + jnp.where(j == 0, extra, 0.0)

    @pl.when((i == 0) & (j == 0))
    def _init():
        out_ref[...] = jnp.zeros((1, 1), jnp.float32)

    out_ref[...] += jnp.full((1, 1), contrib, jnp.float32)


def kernel(predicted_log_probabilities, tgt_tokens):
    n, v = predicted_log_probabilities.shape
    tok_col = tgt_tokens.reshape(n, 1).astype(jnp.float32)
    grid = (n // RB, pl.cdiv(v, CB))
    out = pl.pallas_call(
        _loss_body,
        grid=grid,
        in_specs=[
            pl.BlockSpec((RB, 1), lambda i, j: (i, 0)),
            pl.BlockSpec((RB, CB), lambda i, j: (i, j)),
        ],
        out_specs=pl.BlockSpec((1, 1), lambda i, j: (0, 0)),
        out_shape=jax.ShapeDtypeStruct((1, 1), jnp.float32),
    )(tok_col, predicted_log_probabilities)
    return out[0, 0]
